# baseline (speedup 1.0000x reference)
import functools

import jax
import jax.numpy as jnp
from jax import lax
from jax.experimental import pallas as pl
from jax.experimental.pallas import tpu as pltpu
from jax.experimental.pallas import tpu_sc as plsc

N_NODES = 10000
N_EDGES = 320000
D = 128
NEG = 0.01

NPAD = 10240
NC, NS = 2, 16
NW = NC * NS
EPW = N_EDGES // NW
EPW2 = 10240
E_PAD = NW * EPW2
IR = EPW2 // 128
KSTEPS = EPW2 // 16
NBUF = 5
GROUPS = KSTEPS // NBUF
TRASH_ROW = N_NODES + 100
ROWS_PER_TILE = NPAD // NS
DEG_W = 128

_MESH = plsc.VectorSubcoreMesh(core_axis_name="c", subcore_axis_name="s")


def _zero_vmem_2d(ref, nrows, ncols):
    z16 = jnp.zeros((16,), jnp.float32)
    per_row = ncols // 16

    def body(i, carry):
        r = i // per_row
        c = (i % per_row) * 16
        ref[r, pl.ds(c, 16)] = z16
        return carry

    lax.fori_loop(0, nrows * per_row, body, 0)


@functools.partial(
    pl.kernel,
    out_type=jax.ShapeDtypeStruct((NC, NPAD, DEG_W), jnp.float32),
    mesh=_MESH,
    scratch_types=[
        pltpu.VMEM((IR, 128), jnp.int32),
        pltpu.VMEM((16, DEG_W), jnp.float32),
        pltpu.VMEM_SHARED((NPAD, DEG_W), jnp.float32),
        pltpu.SemaphoreType.DMA,
    ]
    + [pltpu.SemaphoreType.DMA for _ in range(NBUF)],
)
def _sc_deg(dstr_hbm, out_hbm, didx, ones, acc, sem_i, *sem_s):
    cid = lax.axis_index("c")
    sid = lax.axis_index("s")
    wid = sid * NC + cid

    pltpu.async_copy(dstr_hbm.at[wid], didx, sem_i)

    def _dvec(k):
        return didx[k // 8, pl.ds((k % 8) * 16, 16)]

    _zero_vmem_2d(ones, 16, DEG_W)

    def zero_acc(j, carry):
        pltpu.sync_copy(ones, acc.at[pl.ds(sid * ROWS_PER_TILE + j * 16, 16)])
        return carry

    lax.fori_loop(0, ROWS_PER_TILE // 16, zero_acc, 0)

    one16 = jnp.full((16,), 1.0, jnp.float32)

    def fill_ones(i, carry):
        ones[i // 8, pl.ds((i % 8) * 16, 16)] = one16
        return carry

    lax.fori_loop(0, 16 * 8, fill_ones, 0)
    pltpu.make_async_copy(dstr_hbm.at[wid], didx, sem_i).wait()
    plsc.subcore_barrier()

    for b in range(NBUF):
        pltpu.async_copy(ones, acc.at[_dvec(b)], sem_s[b], add=True)

    def outer(g, carry):
        for b in range(NBUF):
            k = g * NBUF + b
            pltpu.make_async_copy(ones, acc.at[_dvec(k)], sem_s[b]).wait()

            @pl.when(g < GROUPS - 1)
            def _():
                pltpu.async_copy(ones, acc.at[_dvec(k + NBUF)], sem_s[b], add=True)

        return carry

    lax.fori_loop(0, GROUPS, outer, 0)
    plsc.subcore_barrier()

    pltpu.sync_copy(
        acc.at[pl.ds(sid * ROWS_PER_TILE, ROWS_PER_TILE)],
        out_hbm.at[cid, pl.ds(sid * ROWS_PER_TILE, ROWS_PER_TILE)],
    )


KD = EPW2 // 128
NIDX = 4
NROW = 2


@functools.partial(
    pl.kernel,
    out_type=jax.ShapeDtypeStruct((NC, NPAD, D), jnp.float32),
    mesh=_MESH,
    scratch_types=[pltpu.VMEM((2, 128), jnp.int32) for _ in range(NIDX)]
    + [pltpu.VMEM((128, D), jnp.float32) for _ in range(NROW)]
    + [pltpu.VMEM_SHARED((NPAD, D), jnp.float32)]
    + [pltpu.SemaphoreType.DMA for _ in range(NIDX + 2 * NROW)],
)
def _sc_scatter(g_hbm, pairs_hbm, out_hbm, *rest):
    idx = rest[:NIDX]
    rows = rest[NIDX:NIDX + NROW]
    acc = rest[NIDX + NROW]
    sem_i = rest[NIDX + NROW + 1:2 * NIDX + NROW + 1]
    sem_g = rest[2 * NIDX + NROW + 1:2 * NIDX + 2 * NROW + 1]
    sem_s = rest[2 * NIDX + 2 * NROW + 1:]

    cid = lax.axis_index("c")
    sid = lax.axis_index("s")
    wid = sid * NC + cid

    for b in range(NIDX):
        pltpu.async_copy(pairs_hbm.at[wid * KD + b], idx[b], sem_i[b])

    _zero_vmem_2d(rows[0], 128, D)

    def zero_acc(j, carry):
        pltpu.sync_copy(rows[0],
                        acc.at[pl.ds(sid * ROWS_PER_TILE + j * 128, 128)])
        return carry

    lax.fori_loop(0, ROWS_PER_TILE // 128, zero_acc, 0)
    plsc.subcore_barrier()

    def outer(g, carry):
        for b in range(NIDX):
            k = g * NIDX + b
            br = b % NROW
            bi2 = (b + 2) % NIDX

            @pl.when(k >= 2)
            def _():
                pltpu.make_async_copy(rows[br], acc.at[idx[bi2].at[1]],
                                      sem_s[br]).wait()

                @pl.when(k + 2 < KD)
                def _():
                    pltpu.async_copy(pairs_hbm.at[wid * KD + k + 2],
                                     idx[bi2], sem_i[bi2])

            pltpu.make_async_copy(pairs_hbm.at[wid * KD + k], idx[b],
                                  sem_i[b]).wait()
            pltpu.async_copy(g_hbm.at[idx[b].at[0]], rows[br], sem_g[br])
            pltpu.make_async_copy(g_hbm.at[idx[b].at[0]], rows[br],
                                  sem_g[br]).wait()
            pltpu.async_copy(rows[br], acc.at[idx[b].at[1]], sem_s[br],
                             add=True)
        return carry

    lax.fori_loop(0, KD // NIDX, outer, 0)
    for k in (KD - 2, KD - 1):
        b = k % NIDX
        pltpu.make_async_copy(rows[k % NROW], acc.at[idx[b].at[1]],
                              sem_s[k % NROW]).wait()
    plsc.subcore_barrier()

    pltpu.sync_copy(
        acc.at[pl.ds(sid * ROWS_PER_TILE, ROWS_PER_TILE)],
        out_hbm.at[cid, pl.ds(sid * ROWS_PER_TILE, ROWS_PER_TILE)],
    )


BLK = 512
GRID = NPAD // BLK


def _dinv_from(deg_ref):
    deg = 1.0 + deg_ref[0, :, 0:1] + deg_ref[1, :, 0:1]
    return lax.rsqrt(deg)


def _tc_pre_body(x_ref, w_ref, deg_ref, h_ref, g_ref):
    h = jnp.dot(x_ref[...], w_ref[...], preferred_element_type=jnp.float32)
    dinv = _dinv_from(deg_ref)
    h_ref[...] = h
    g_ref[...] = h * dinv


def _tc_mid_body(s_ref, h_ref, deg_ref, b_ref, w_ref, hn_ref, gn_ref):
    dinv = _dinv_from(deg_ref)
    h = h_ref[...]
    v = dinv * (s_ref[0] + s_ref[1]) + (dinv * dinv) * h + b_ref[...]
    a = jnp.where(v >= 0, v, NEG * v)
    hn = jnp.dot(a, w_ref[...], preferred_element_type=jnp.float32)
    hn_ref[...] = hn
    gn_ref[...] = hn * dinv


def _tc_post_body(s_ref, h_ref, deg_ref, b_ref, o_ref):
    dinv = _dinv_from(deg_ref)
    h = h_ref[...]
    v = dinv * (s_ref[0] + s_ref[1]) + (dinv * dinv) * h + b_ref[...]
    o_ref[...] = jnp.where(v >= 0, v, NEG * v)


_ROWBLK = pl.BlockSpec((BLK, D), lambda i: (i, 0))
_WSPEC = pl.BlockSpec((D, D), lambda i: (0, 0))
_DEGSPEC = pl.BlockSpec((NC, BLK, DEG_W), lambda i: (0, i, 0))
_SSPEC = pl.BlockSpec((NC, BLK, D), lambda i: (0, i, 0))
_BSPEC = pl.BlockSpec((1, D), lambda i: (0, 0))
_F32ROW = jax.ShapeDtypeStruct((NPAD, D), jnp.float32)

_tc_pre = pl.pallas_call(
    _tc_pre_body,
    grid=(GRID,),
    in_specs=[_ROWBLK, _WSPEC, _DEGSPEC],
    out_specs=[_ROWBLK, _ROWBLK],
    out_shape=[_F32ROW, _F32ROW],
)

_tc_mid = pl.pallas_call(
    _tc_mid_body,
    grid=(GRID,),
    in_specs=[_SSPEC, _ROWBLK, _DEGSPEC, _BSPEC, _WSPEC],
    out_specs=[_ROWBLK, _ROWBLK],
    out_shape=[_F32ROW, _F32ROW],
)

_tc_post = pl.pallas_call(
    _tc_post_body,
    grid=(GRID,),
    in_specs=[_SSPEC, _ROWBLK, _DEGSPEC, _BSPEC],
    out_specs=_ROWBLK,
    out_shape=_F32ROW,
)


def kernel(x, edge_index, W1, b1, W2, b2, W3, b3):
    pad = E_PAD - N_EDGES
    src_s = jnp.concatenate(
        [edge_index[0], jnp.zeros((pad,), jnp.int32)]).reshape(NW, IR, 128)
    dst_s = jnp.concatenate(
        [edge_index[1], jnp.full((pad,), TRASH_ROW, jnp.int32)]).reshape(NW, IR, 128)
    pairs = jnp.stack([src_s, dst_s], axis=2).reshape(NW * KD, 2, 128)
    x_pad = jnp.zeros((NPAD, D), jnp.float32).at[:N_NODES].set(x)
    b1r = b1.reshape(1, D)
    b2r = b2.reshape(1, D)
    b3r = b3.reshape(1, D)

    degp = _sc_deg(dst_s)
    h1, g1 = _tc_pre(x_pad, W1, degp)
    s1 = _sc_scatter(g1, pairs)
    h2, g2 = _tc_mid(s1, h1, degp, b1r, W2)
    s2 = _sc_scatter(g2, pairs)
    h3, g3 = _tc_mid(s2, h2, degp, b2r, W3)
    s3 = _sc_scatter(g3, pairs)
    out = _tc_post(s3, h3, degp, b3r)
    return out[:N_NODES]


# speedup vs baseline: 2.7519x; 2.7519x over previous
import functools

import jax
import jax.numpy as jnp
from jax import lax
from jax.experimental import pallas as pl
from jax.experimental.pallas import tpu as pltpu
from jax.experimental.pallas import tpu_sc as plsc

N_NODES = 10000
N_EDGES = 320000
D = 128
NEG = 0.01

NPAD = 10240
NC, NS = 2, 16
NW = NC * NS
EPW = N_EDGES // NW
EPW2 = 10240
E_PAD = NW * EPW2
IR = EPW2 // 128
KSTEPS = EPW2 // 16
NBUF = 5
GROUPS = KSTEPS // NBUF
TRASH_ROW = N_NODES + 100
ROWS_PER_TILE = NPAD // NS
DEG_W = 128

_MESH = plsc.VectorSubcoreMesh(core_axis_name="c", subcore_axis_name="s")


def _zero_vmem_2d(ref, nrows, ncols):
    z16 = jnp.zeros((16,), jnp.float32)
    per_row = ncols // 16

    def body(i, carry):
        r = i // per_row
        c = (i % per_row) * 16
        ref[r, pl.ds(c, 16)] = z16
        return carry

    lax.fori_loop(0, nrows * per_row, body, 0)


@functools.partial(
    pl.kernel,
    out_type=jax.ShapeDtypeStruct((NC, NPAD, DEG_W), jnp.float32),
    mesh=_MESH,
    scratch_types=[
        pltpu.VMEM((IR, 128), jnp.int32),
        pltpu.VMEM((16, DEG_W), jnp.float32),
        pltpu.VMEM_SHARED((NPAD, DEG_W), jnp.float32),
        pltpu.SemaphoreType.DMA,
    ]
    + [pltpu.SemaphoreType.DMA for _ in range(NBUF)],
)
def _sc_deg(dstr_hbm, out_hbm, didx, ones, acc, sem_i, *sem_s):
    cid = lax.axis_index("c")
    sid = lax.axis_index("s")
    wid = sid * NC + cid

    pltpu.async_copy(dstr_hbm.at[wid], didx, sem_i)

    def _dvec(k):
        return didx[k // 8, pl.ds((k % 8) * 16, 16)]

    _zero_vmem_2d(ones, 16, DEG_W)

    def zero_acc(j, carry):
        pltpu.sync_copy(ones, acc.at[pl.ds(sid * ROWS_PER_TILE + j * 16, 16)])
        return carry

    lax.fori_loop(0, ROWS_PER_TILE // 16, zero_acc, 0)

    one16 = jnp.full((16,), 1.0, jnp.float32)

    def fill_ones(i, carry):
        ones[i // 8, pl.ds((i % 8) * 16, 16)] = one16
        return carry

    lax.fori_loop(0, 16 * 8, fill_ones, 0)
    pltpu.make_async_copy(dstr_hbm.at[wid], didx, sem_i).wait()
    plsc.subcore_barrier()

    for b in range(NBUF):
        pltpu.async_copy(ones, acc.at[_dvec(b)], sem_s[b], add=True)

    def outer(g, carry):
        for b in range(NBUF):
            k = g * NBUF + b
            pltpu.make_async_copy(ones, acc.at[_dvec(k)], sem_s[b]).wait()

            @pl.when(g < GROUPS - 1)
            def _():
                pltpu.async_copy(ones, acc.at[_dvec(k + NBUF)], sem_s[b], add=True)

        return carry

    lax.fori_loop(0, GROUPS, outer, 0)
    plsc.subcore_barrier()

    pltpu.sync_copy(
        acc.at[pl.ds(sid * ROWS_PER_TILE, ROWS_PER_TILE)],
        out_hbm.at[cid, pl.ds(sid * ROWS_PER_TILE, ROWS_PER_TILE)],
    )


KD = EPW2 // 128
NIDX = 4
NROW = 2


@functools.partial(
    pl.kernel,
    out_type=jax.ShapeDtypeStruct((NC, NPAD, D), jnp.float32),
    mesh=_MESH,
    scratch_types=[pltpu.VMEM((2, 128), jnp.int32) for _ in range(NIDX)]
    + [pltpu.VMEM((128, D), jnp.float32) for _ in range(NROW)]
    + [pltpu.VMEM_SHARED((NPAD, D), jnp.float32)]
    + [pltpu.SemaphoreType.DMA for _ in range(NIDX + 2 * NROW)],
)
def _sc_scatter(g_hbm, pairs_hbm, out_hbm, *rest):
    idx = rest[:NIDX]
    rows = rest[NIDX:NIDX + NROW]
    acc = rest[NIDX + NROW]
    sem_i = rest[NIDX + NROW + 1:2 * NIDX + NROW + 1]
    sem_g = rest[2 * NIDX + NROW + 1:2 * NIDX + 2 * NROW + 1]
    sem_s = rest[2 * NIDX + 2 * NROW + 1:]

    cid = lax.axis_index("c")
    sid = lax.axis_index("s")
    wid = sid * NC + cid

    for b in range(NIDX):
        pltpu.async_copy(pairs_hbm.at[wid * KD + b], idx[b], sem_i[b])

    _zero_vmem_2d(rows[0], 128, D)

    def zero_acc(j, carry):
        pltpu.sync_copy(rows[0],
                        acc.at[pl.ds(sid * ROWS_PER_TILE + j * 128, 128)])
        return carry

    lax.fori_loop(0, ROWS_PER_TILE // 128, zero_acc, 0)
    plsc.subcore_barrier()

    def outer(g, carry):
        for b in range(NIDX):
            k = g * NIDX + b
            br = b % NROW
            bi2 = (b + 2) % NIDX

            @pl.when(k >= 2)
            def _():
                pltpu.make_async_copy(rows[br], acc.at[idx[bi2].at[1]],
                                      sem_s[br]).wait()

                @pl.when(k + 2 < KD)
                def _():
                    pltpu.async_copy(pairs_hbm.at[wid * KD + k + 2],
                                     idx[bi2], sem_i[bi2])

            pltpu.make_async_copy(pairs_hbm.at[wid * KD + k], idx[b],
                                  sem_i[b]).wait()
            pltpu.async_copy(g_hbm.at[idx[b].at[0]], rows[br], sem_g[br])
            pltpu.make_async_copy(g_hbm.at[idx[b].at[0]], rows[br],
                                  sem_g[br]).wait()
            pltpu.async_copy(rows[br], acc.at[idx[b].at[1]], sem_s[br],
                             add=True)
        return carry

    lax.fori_loop(0, KD // NIDX, outer, 0)
    for k in (KD - 2, KD - 1):
        b = k % NIDX
        pltpu.make_async_copy(rows[k % NROW], acc.at[idx[b].at[1]],
                              sem_s[k % NROW]).wait()
    plsc.subcore_barrier()

    pltpu.sync_copy(
        acc.at[pl.ds(sid * ROWS_PER_TILE, ROWS_PER_TILE)],
        out_hbm.at[cid, pl.ds(sid * ROWS_PER_TILE, ROWS_PER_TILE)],
    )


BLK = 512
GRID = NPAD // BLK


def _dinv_from(deg_ref):
    deg = 1.0 + deg_ref[0, :, 0:1] + deg_ref[1, :, 0:1]
    return lax.rsqrt(deg)


def _tc_pre_body(x_ref, w_ref, deg_ref, h_ref, g_ref):
    h = jnp.dot(x_ref[...], w_ref[...], preferred_element_type=jnp.float32)
    dinv = _dinv_from(deg_ref)
    h_ref[...] = h
    g_ref[...] = h * dinv


def _tc_mid_body(s_ref, h_ref, deg_ref, b_ref, w_ref, hn_ref, gn_ref):
    dinv = _dinv_from(deg_ref)
    h = h_ref[...]
    v = dinv * (s_ref[0] + s_ref[1]) + (dinv * dinv) * h + b_ref[...]
    a = jnp.where(v >= 0, v, NEG * v)
    hn = jnp.dot(a, w_ref[...], preferred_element_type=jnp.float32)
    hn_ref[...] = hn
    gn_ref[...] = hn * dinv


def _tc_post_body(s_ref, h_ref, deg_ref, b_ref, o_ref):
    dinv = _dinv_from(deg_ref)
    h = h_ref[...]
    v = dinv * (s_ref[0] + s_ref[1]) + (dinv * dinv) * h + b_ref[...]
    o_ref[...] = jnp.where(v >= 0, v, NEG * v)


_ROWBLK = pl.BlockSpec((BLK, D), lambda i: (i, 0))
_WSPEC = pl.BlockSpec((D, D), lambda i: (0, 0))
_DEGSPEC = pl.BlockSpec((NC, BLK, DEG_W), lambda i: (0, i, 0))
_SSPEC = pl.BlockSpec((NC, BLK, D), lambda i: (0, i, 0))
_BSPEC = pl.BlockSpec((1, D), lambda i: (0, 0))
_F32ROW = jax.ShapeDtypeStruct((NPAD, D), jnp.float32)

_tc_pre = pl.pallas_call(
    _tc_pre_body,
    grid=(GRID,),
    in_specs=[_ROWBLK, _WSPEC, _DEGSPEC],
    out_specs=[_ROWBLK, _ROWBLK],
    out_shape=[_F32ROW, _F32ROW],
)

_tc_mid = pl.pallas_call(
    _tc_mid_body,
    grid=(GRID,),
    in_specs=[_SSPEC, _ROWBLK, _DEGSPEC, _BSPEC, _WSPEC],
    out_specs=[_ROWBLK, _ROWBLK],
    out_shape=[_F32ROW, _F32ROW],
)

_tc_post = pl.pallas_call(
    _tc_post_body,
    grid=(GRID,),
    in_specs=[_SSPEC, _ROWBLK, _DEGSPEC, _BSPEC],
    out_specs=_ROWBLK,
    out_shape=_F32ROW,
)


def kernel(x, edge_index, W1, b1, W2, b2, W3, b3):
    pad = E_PAD - N_EDGES
    tpos = jnp.arange(pad, dtype=jnp.int32)
    src_s = jnp.concatenate(
        [edge_index[0], tpos % N_NODES]).reshape(NW, IR, 128)
    dst_s = jnp.concatenate(
        [edge_index[1], N_NODES + tpos % (NPAD - N_NODES)]).reshape(NW, IR, 128)
    pairs = jnp.stack([src_s, dst_s], axis=2).reshape(NW * KD, 2, 128)
    x_pad = jnp.zeros((NPAD, D), jnp.float32).at[:N_NODES].set(x)
    b1r = b1.reshape(1, D)
    b2r = b2.reshape(1, D)
    b3r = b3.reshape(1, D)

    degp = _sc_deg(dst_s)
    h1, g1 = _tc_pre(x_pad, W1, degp)
    s1 = _sc_scatter(g1, pairs)
    h2, g2 = _tc_mid(s1, h1, degp, b1r, W2)
    s2 = _sc_scatter(g2, pairs)
    h3, g3 = _tc_mid(s2, h2, degp, b2r, W3)
    s3 = _sc_scatter(g3, pairs)
    out = _tc_post(s3, h3, degp, b3r)
    return out[:N_NODES]


# speedup vs baseline: 2.7557x; 1.0014x over previous
import functools

import jax
import jax.numpy as jnp
from jax import lax
from jax.experimental import pallas as pl
from jax.experimental.pallas import tpu as pltpu
from jax.experimental.pallas import tpu_sc as plsc

N_NODES = 10000
N_EDGES = 320000
D = 128
NEG = 0.01

NPAD = 10240
NC, NS = 2, 16
NW = NC * NS
EPW = N_EDGES // NW
EPW2 = 10240
E_PAD = NW * EPW2
IR = EPW2 // 128
KSTEPS = EPW2 // 16
NBUF = 5
GROUPS = KSTEPS // NBUF
ROWS_PER_TILE = NPAD // NS
DEG_W = 128

_MESH = plsc.VectorSubcoreMesh(core_axis_name="c", subcore_axis_name="s")


def _zero_vmem_2d(ref, nrows, ncols):
    z16 = jnp.zeros((16,), jnp.float32)
    per_row = ncols // 16

    def body(i, carry):
        r = i // per_row
        c = (i % per_row) * 16
        ref[r, pl.ds(c, 16)] = z16
        return carry

    lax.fori_loop(0, nrows * per_row, body, 0)


@functools.partial(
    pl.kernel,
    out_type=jax.ShapeDtypeStruct((NC, NPAD, DEG_W), jnp.float32),
    mesh=_MESH,
    scratch_types=[
        pltpu.VMEM((IR, 128), jnp.int32),
        pltpu.VMEM((16, DEG_W), jnp.float32),
        pltpu.VMEM_SHARED((NPAD, DEG_W), jnp.float32),
        pltpu.SemaphoreType.DMA,
    ]
    + [pltpu.SemaphoreType.DMA for _ in range(NBUF)],
)
def _sc_deg(dstr_hbm, out_hbm, didx, ones, acc, sem_i, *sem_s):
    cid = lax.axis_index("c")
    sid = lax.axis_index("s")
    wid = sid * NC + cid

    pltpu.async_copy(dstr_hbm.at[wid], didx, sem_i)

    def _dvec(k):
        return didx[k // 8, pl.ds((k % 8) * 16, 16)]

    _zero_vmem_2d(ones, 16, DEG_W)

    def zero_acc(j, carry):
        pltpu.sync_copy(ones, acc.at[pl.ds(sid * ROWS_PER_TILE + j * 16, 16)])
        return carry

    lax.fori_loop(0, ROWS_PER_TILE // 16, zero_acc, 0)

    one16 = jnp.full((16,), 1.0, jnp.float32)

    def fill_ones(i, carry):
        ones[i // 8, pl.ds((i % 8) * 16, 16)] = one16
        return carry

    lax.fori_loop(0, 16 * 8, fill_ones, 0)
    pltpu.make_async_copy(dstr_hbm.at[wid], didx, sem_i).wait()
    plsc.subcore_barrier()

    for b in range(NBUF):
        pltpu.async_copy(ones, acc.at[_dvec(b)], sem_s[b], add=True)

    def outer(g, carry):
        for b in range(NBUF):
            k = g * NBUF + b
            pltpu.make_async_copy(ones, acc.at[_dvec(k)], sem_s[b]).wait()

            @pl.when(g < GROUPS - 1)
            def _():
                pltpu.async_copy(ones, acc.at[_dvec(k + NBUF)], sem_s[b], add=True)

        return carry

    lax.fori_loop(0, GROUPS, outer, 0)
    plsc.subcore_barrier()

    pltpu.sync_copy(
        acc.at[pl.ds(sid * ROWS_PER_TILE, ROWS_PER_TILE)],
        out_hbm.at[cid, pl.ds(sid * ROWS_PER_TILE, ROWS_PER_TILE)],
    )


KD = EPW2 // 128
NIDX = 4
NROW = 2


@functools.partial(
    pl.kernel,
    out_type=jax.ShapeDtypeStruct((NC, NPAD, D), jnp.float32),
    mesh=_MESH,
    scratch_types=[pltpu.VMEM((2, 128), jnp.int32) for _ in range(NIDX)]
    + [pltpu.VMEM((128, D), jnp.float32) for _ in range(NROW)]
    + [pltpu.VMEM_SHARED((NPAD, D), jnp.float32)]
    + [pltpu.SemaphoreType.DMA for _ in range(NIDX + 2 * NROW)],
)
def _sc_scatter(g_hbm, pairs_hbm, out_hbm, *rest):
    idx = rest[:NIDX]
    rows = rest[NIDX:NIDX + NROW]
    acc = rest[NIDX + NROW]
    sem_i = rest[NIDX + NROW + 1:2 * NIDX + NROW + 1]
    sem_g = rest[2 * NIDX + NROW + 1:2 * NIDX + 2 * NROW + 1]
    sem_s = rest[2 * NIDX + 2 * NROW + 1:]

    cid = lax.axis_index("c")
    sid = lax.axis_index("s")
    wid = sid * NC + cid

    for b in range(NIDX):
        pltpu.async_copy(pairs_hbm.at[wid * KD + b], idx[b], sem_i[b])

    _zero_vmem_2d(rows[0], 128, D)

    def zero_acc(j, carry):
        pltpu.sync_copy(rows[0],
                        acc.at[pl.ds(sid * ROWS_PER_TILE + j * 128, 128)])
        return carry

    lax.fori_loop(0, ROWS_PER_TILE // 128, zero_acc, 0)
    plsc.subcore_barrier()

    def outer(g, carry):
        for b in range(NIDX):
            k = g * NIDX + b
            br = b % NROW
            bi2 = (b + 2) % NIDX

            @pl.when(k >= 2)
            def _():
                pltpu.make_async_copy(rows[br], acc.at[idx[bi2].at[1]],
                                      sem_s[br]).wait()

                @pl.when(k + 2 < KD)
                def _():
                    pltpu.async_copy(pairs_hbm.at[wid * KD + k + 2],
                                     idx[bi2], sem_i[bi2])

            pltpu.make_async_copy(pairs_hbm.at[wid * KD + k], idx[b],
                                  sem_i[b]).wait()
            pltpu.async_copy(g_hbm.at[idx[b].at[0]], rows[br], sem_g[br])
            pltpu.make_async_copy(g_hbm.at[idx[b].at[0]], rows[br],
                                  sem_g[br]).wait()
            pltpu.async_copy(rows[br], acc.at[idx[b].at[1]], sem_s[br],
                             add=True)
        return carry

    lax.fori_loop(0, KD // NIDX, outer, 0)
    for k in (KD - 2, KD - 1):
        b = k % NIDX
        pltpu.make_async_copy(rows[k % NROW], acc.at[idx[b].at[1]],
                              sem_s[k % NROW]).wait()
    plsc.subcore_barrier()

    pltpu.sync_copy(
        acc.at[pl.ds(sid * ROWS_PER_TILE, ROWS_PER_TILE)],
        out_hbm.at[cid, pl.ds(sid * ROWS_PER_TILE, ROWS_PER_TILE)],
    )


BLK = 512
GRID = NPAD // BLK


def _dinv_from(deg_ref):
    deg = 1.0 + deg_ref[0, :, 0:1] + deg_ref[1, :, 0:1]
    return lax.rsqrt(deg)


def _tc_pre_body(x_ref, w_ref, deg_ref, h_ref, g_ref):
    h = jnp.dot(x_ref[...], w_ref[...], preferred_element_type=jnp.float32)
    dinv = _dinv_from(deg_ref)
    h_ref[...] = h
    g_ref[...] = h * dinv


def _tc_mid_body(s_ref, h_ref, deg_ref, b_ref, w_ref, hn_ref, gn_ref):
    dinv = _dinv_from(deg_ref)
    h = h_ref[...]
    v = dinv * (s_ref[0] + s_ref[1]) + (dinv * dinv) * h + b_ref[...]
    a = jnp.where(v >= 0, v, NEG * v)
    hn = jnp.dot(a, w_ref[...], preferred_element_type=jnp.float32)
    hn_ref[...] = hn
    gn_ref[...] = hn * dinv


def _tc_post_body(s_ref, h_ref, deg_ref, b_ref, o_ref):
    dinv = _dinv_from(deg_ref)
    h = h_ref[...]
    v = dinv * (s_ref[0] + s_ref[1]) + (dinv * dinv) * h + b_ref[...]
    o_ref[...] = jnp.where(v >= 0, v, NEG * v)


_ROWBLK = pl.BlockSpec((BLK, D), lambda i: (i, 0))
_WSPEC = pl.BlockSpec((D, D), lambda i: (0, 0))
_DEGSPEC = pl.BlockSpec((NC, BLK, DEG_W), lambda i: (0, i, 0))
_SSPEC = pl.BlockSpec((NC, BLK, D), lambda i: (0, i, 0))
_BSPEC = pl.BlockSpec((1, D), lambda i: (0, 0))
_F32ROW = jax.ShapeDtypeStruct((NPAD, D), jnp.float32)

_tc_pre = pl.pallas_call(
    _tc_pre_body,
    grid=(GRID,),
    in_specs=[_ROWBLK, _WSPEC, _DEGSPEC],
    out_specs=[_ROWBLK, _ROWBLK],
    out_shape=[_F32ROW, _F32ROW],
)

_tc_mid = pl.pallas_call(
    _tc_mid_body,
    grid=(GRID,),
    in_specs=[_SSPEC, _ROWBLK, _DEGSPEC, _BSPEC, _WSPEC],
    out_specs=[_ROWBLK, _ROWBLK],
    out_shape=[_F32ROW, _F32ROW],
)

_tc_post = pl.pallas_call(
    _tc_post_body,
    grid=(GRID,),
    in_specs=[_SSPEC, _ROWBLK, _DEGSPEC, _BSPEC],
    out_specs=_ROWBLK,
    out_shape=_F32ROW,
)


def kernel(x, edge_index, W1, b1, W2, b2, W3, b3):
    pad = E_PAD - N_EDGES
    tpos = jnp.arange(pad, dtype=jnp.int32)
    src_s = jnp.concatenate(
        [edge_index[0], tpos % N_NODES]).reshape(NW, IR, 128)
    dst_s = jnp.concatenate(
        [edge_index[1], N_NODES + tpos % (NPAD - N_NODES)]).reshape(NW, IR, 128)
    pairs = jnp.stack([src_s, dst_s], axis=2).reshape(NW * KD, 2, 128)
    x_pad = jnp.zeros((NPAD, D), jnp.float32).at[:N_NODES].set(x)
    b1r = b1.reshape(1, D)
    b2r = b2.reshape(1, D)
    b3r = b3.reshape(1, D)

    degp = _sc_deg(dst_s)
    h1, g1 = _tc_pre(x_pad, W1, degp)
    s1 = _sc_scatter(g1, pairs)
    h2, g2 = _tc_mid(s1, h1, degp, b1r, W2)
    s2 = _sc_scatter(g2, pairs)
    h3, g3 = _tc_mid(s2, h2, degp, b2r, W3)
    s3 = _sc_scatter(g3, pairs)
    out = _tc_post(s3, h3, degp, b3r)
    return out[:N_NODES]
